# baseline (speedup 1.0000x reference)
import functools

import jax
import jax.numpy as jnp
from jax import lax
from jax.experimental import pallas as pl
from jax.experimental.pallas import tpu as pltpu
from jax.experimental.pallas import tpu_sc as plsc

N, D, H, O, E, TOPK = 2048, 1024, 2048, 1024, 8, 2
S = N * TOPK
TN = 256
P = S + E * TN
NTT = P // TN
G = 4
HC = H // G
EP = 128
TG = 256
NTG = N // TG

NW = 32
_RC = S // NW
_RCH = 32
_RNC = _RC // _RCH
_CT = N // NW
_CCH = 32


def _gating_body(x_ref, wg_ref, bg_ref, probs_ref, idx_ref, wn_ref, loss_ref,
                 acc_ref):
    i = pl.program_id(0)
    xt = x_ref[...]
    logits = jax.lax.dot_general(
        xt, wg_ref[...], (((1,), (0,)), ((), ())),
        preferred_element_type=jnp.float32) + bg_ref[...]
    col = jax.lax.broadcasted_iota(jnp.int32, (TG, EP), 1)
    valid = col < E
    logits = jnp.where(valid, logits, -jnp.inf)
    m = jnp.max(logits, axis=1, keepdims=True)
    ex = jnp.exp(logits - m)
    s = jnp.sum(ex, axis=1, keepdims=True)
    probs = ex / s
    probs_ref[...] = probs

    p1 = jnp.max(probs, axis=1, keepdims=True)
    i1 = jnp.min(jnp.where((probs == p1) & valid, col, EP), axis=1,
                 keepdims=True)
    one1 = col == i1
    probs_m = jnp.where(one1, -1.0, probs)
    p2 = jnp.max(probs_m, axis=1, keepdims=True)
    i2 = jnp.min(jnp.where((probs_m == p2) & valid, col, EP), axis=1,
                 keepdims=True)
    denom = p1 + p2
    idx_ref[...] = jnp.where(col == 0, i1, jnp.where(col == 1, i2, 0))
    wn_ref[...] = jnp.where(col == 0, p1 / denom,
                            jnp.where(col == 1, p2 / denom, 0.0))

    part = jnp.sum(probs, axis=0, keepdims=True)
    @pl.when(i == 0)
    def _():
        acc_ref[...] = part
    @pl.when(i > 0)
    def _():
        acc_ref[...] += part
    @pl.when(i == NTG - 1)
    def _():
        usage = acc_ref[...] / N
        lane = jax.lax.broadcasted_iota(jnp.int32, (1, EP), 1)
        uni = jnp.float32(1.0 / E)
        term = uni * (jnp.log(uni) - jnp.log(usage + 1e-8))
        loss_ref[...] = jnp.sum(jnp.where(lane < E, term, 0.0), axis=1,
                                keepdims=True) * 0.01


def _gating(x, Wg, bg):
    wgp = jnp.zeros((D, EP), jnp.float32).at[:, :E].set(Wg.T)
    bgp = jnp.zeros((1, EP), jnp.float32).at[0, :E].set(bg)
    return pl.pallas_call(
        _gating_body,
        grid=(NTG,),
        in_specs=[
            pl.BlockSpec((TG, D), lambda i: (i, 0)),
            pl.BlockSpec((D, EP), lambda i: (0, 0)),
            pl.BlockSpec((1, EP), lambda i: (0, 0)),
        ],
        out_specs=[
            pl.BlockSpec((TG, EP), lambda i: (i, 0)),
            pl.BlockSpec((TG, EP), lambda i: (i, 0)),
            pl.BlockSpec((TG, EP), lambda i: (i, 0)),
            pl.BlockSpec((1, 1), lambda i: (0, 0)),
        ],
        out_shape=[
            jax.ShapeDtypeStruct((N, EP), jnp.float32),
            jax.ShapeDtypeStruct((N, EP), jnp.int32),
            jax.ShapeDtypeStruct((N, EP), jnp.float32),
            jax.ShapeDtypeStruct((1, 1), jnp.float32),
        ],
        scratch_shapes=[pltpu.VMEM((1, EP), jnp.float32)],
    )(x, wgp, bgp)


def _count_body(es_hbm, cnt_hbm, es_v, cnt_v):
    wid = lax.axis_index("s") * 2 + lax.axis_index("c")
    base = wid * _RC
    lane = lax.iota(jnp.int32, 16)
    pltpu.sync_copy(es_hbm.at[pl.ds(base, _RC)], es_v)
    counts = [jnp.int32(0)] * E
    for v in range(_RC // 16):
        ids = es_v[pl.ds(v * 16, 16)]
        for e in range(E):
            counts[e] = counts[e] + jnp.sum(
                jnp.where(ids == e, 1, 0).astype(jnp.int32))
    cvec = jnp.zeros((16,), jnp.int32)
    for e in range(E):
        cvec = jnp.where(lane == e, counts[e], cvec)
    cnt_v[...] = cvec
    pltpu.sync_copy(cnt_v, cnt_hbm.at[wid])


def _count(es):
    mesh = plsc.VectorSubcoreMesh(core_axis_name="c", subcore_axis_name="s",
                                  num_cores=2, num_subcores=16)
    f = functools.partial(
        pl.kernel,
        out_type=jax.ShapeDtypeStruct((NW, 16), jnp.int32),
        mesh=mesh,
        scratch_types=[
            pltpu.VMEM((_RC,), jnp.int32),
            pltpu.VMEM((16,), jnp.int32),
        ],
        compiler_params=pltpu.CompilerParams(needs_layout_passes=False),
    )
    return f(_count_body)(es)


def _assign_body(es_hbm, ws_hbm, cnt_hbm, x_hbm, pw_hbm, pos2_hbm,
                 eot_hbm, meta_hbm, xg_hbm, es_v, ws_v, posb, tokb, idxb,
                 allc_v, eotb, metab, rb0, rb1, semg0, semg1, semw0, semw1,
                 sems):
    wid = lax.axis_index("s") * 2 + lax.axis_index("c")
    base = wid * _RC
    lane = lax.iota(jnp.int32, 16)
    zi = jnp.zeros((16,), jnp.int32)

    pltpu.sync_copy(es_hbm.at[pl.ds(base, _RC)], es_v)
    for c in range(_RNC):
        pltpu.sync_copy(ws_hbm.at[pl.ds(base + c * _RCH, _RCH)], ws_v.at[c])
    pltpu.sync_copy(cnt_hbm, allc_v)

    tvec = zi
    pvec = zi
    for w2 in range(NW):
        row = allc_v[w2]
        tvec = tvec + row
        pvec = pvec + jnp.where(jnp.int32(w2) < wid, row, zi)
    tot = [tvec[e] for e in range(E)]
    pref = [pvec[e] for e in range(E)]
    ntile = [(tot[e] + TN - 1) // TN for e in range(E)]
    base_tile = []
    bt = jnp.int32(0)
    for e in range(E):
        base_tile.append(bt)
        bt = bt + ntile[e]
    ntt_total = bt
    start = [base_tile[e] * TN + pref[e] for e in range(E)]

    for v in range(_RC // 16):
        ids = es_v[pl.ds(v * 16, 16)]
        pos = zi
        for e in range(E):
            msk = ids == e
            ones = jnp.where(msk, 1, 0).astype(jnp.int32)
            incl = plsc.cumsum(ones)
            pos = jnp.where(msk, start[e] + incl - 1, pos)
            start[e] = start[e] + jnp.sum(ones)
        slot = base + v * 16 + lane
        c, r = v // 2, (v % 2) * 16
        posb[c, pl.ds(r, 16)] = pos
        tokb[c, pl.ds(r, 16)] = slot >> 1
        idxb[c, pl.ds(r, 16)] = (slot & 1) * N + (slot >> 1)

    small = []
    for c in range(_RNC):
        small.append(pltpu.async_copy(ws_v.at[c], pw_hbm.at[posb.at[c]],
                                      sems))
        small.append(pltpu.async_copy(posb.at[c], pos2_hbm.at[idxb.at[c]],
                                      sems))

    rbufs = (rb0, rb1)
    gsems = (semg0, semg1)
    wsems = (semw0, semw1)
    g = {}
    w = {}
    g[0] = pltpu.async_copy(x_hbm.at[tokb.at[0]], rb0, semg0)
    g[1] = pltpu.async_copy(x_hbm.at[tokb.at[1]], rb1, semg1)
    for c in range(_RNC):
        g[c].wait()
        w[c] = pltpu.async_copy(rbufs[c % 2], xg_hbm.at[posb.at[c]],
                                wsems[c % 2])
        if c + 2 < _RNC:
            w[c].wait()
            g[c + 2] = pltpu.async_copy(x_hbm.at[tokb.at[c + 2]],
                                        rbufs[c % 2], gsems[c % 2])
    for c in range(max(0, _RNC - 2), _RNC):
        w[c].wait()
    for cp in small:
        cp.wait()

    @pl.when(wid == 0)
    def _():
        for gg in range(2):
            tid = gg * 16 + lane
            eotv = zi
            for e in range(E):
                msk = (tid >= base_tile[e]) & (tid < base_tile[e] + ntile[e])
                eotv = jnp.where(msk, e, eotv)
            eotv = jnp.where(tid >= ntt_total, E - 1, eotv)
            eotb[pl.ds(gg * 16, 16)] = eotv
        metab[...] = jnp.where(lane == 0, ntt_total, 0)
        pltpu.sync_copy(eotb, eot_hbm)
        pltpu.sync_copy(metab, meta_hbm)


def _assign(es, ws, cnt, x):
    mesh = plsc.VectorSubcoreMesh(core_axis_name="c", subcore_axis_name="s",
                                  num_cores=2, num_subcores=16)
    f = functools.partial(
        pl.kernel,
        out_type=[
            jax.ShapeDtypeStruct((P,), jnp.float32),
            jax.ShapeDtypeStruct((2 * N,), jnp.int32),
            jax.ShapeDtypeStruct((32,), jnp.int32),
            jax.ShapeDtypeStruct((16,), jnp.int32),
            jax.ShapeDtypeStruct((P, D), jnp.float32),
        ],
        mesh=mesh,
        scratch_types=[
            pltpu.VMEM((_RC,), jnp.int32),
            pltpu.VMEM((_RNC, _RCH), jnp.float32),
            pltpu.VMEM((_RNC, _RCH), jnp.int32),
            pltpu.VMEM((_RNC, _RCH), jnp.int32),
            pltpu.VMEM((_RNC, _RCH), jnp.int32),
            pltpu.VMEM((NW, 16), jnp.int32),
            pltpu.VMEM((32,), jnp.int32),
            pltpu.VMEM((16,), jnp.int32),
            pltpu.VMEM((_RCH, D), jnp.float32),
            pltpu.VMEM((_RCH, D), jnp.float32),
            pltpu.SemaphoreType.DMA,
            pltpu.SemaphoreType.DMA,
            pltpu.SemaphoreType.DMA,
            pltpu.SemaphoreType.DMA,
            pltpu.SemaphoreType.DMA,
        ],
        compiler_params=pltpu.CompilerParams(needs_layout_passes=False),
    )
    return f(_assign_body)(es, ws, cnt, x)


def _gffn_body(eot_ref, meta_ref, xg_ref, w1_ref, b1_ref, w2_ref, b2_ref,
               w3_ref, b3_ref, pw_ref, out_ref, h1_ref, w1b_ref, w2b_ref,
               w3b_ref):
    i = pl.program_id(0)
    g = pl.program_id(1)
    nact = meta_ref[0]
    new_e = jnp.logical_or(i == 0, eot_ref[i] != eot_ref[jnp.maximum(i - 1, 0)])

    @pl.when(i < nact)
    def _():
        @pl.when(new_e)
        def _():
            w2b_ref[pl.ds(g * HC, HC), :] = w2_ref[0].astype(jnp.bfloat16)
            w3b_ref[:, pl.ds(g * HC, HC)] = w3_ref[0].astype(jnp.bfloat16)
            @pl.when(g == 0)
            def _():
                w1b_ref[...] = w1_ref[0].astype(jnp.bfloat16)
        @pl.when(g == 0)
        def _():
            h1 = jax.lax.dot_general(
                xg_ref[...].astype(jnp.bfloat16), w1b_ref[...],
                (((1,), (1,)), ((), ())),
                preferred_element_type=jnp.float32) + b1_ref[0]
            h1_ref[...] = jnp.maximum(h1, 0.0).astype(jnp.bfloat16)
        h2c = jax.lax.dot_general(
            h1_ref[...], w2b_ref[pl.ds(g * HC, HC), :],
            (((1,), (1,)), ((), ())),
            preferred_element_type=jnp.float32) + b2_ref[0]
        h2c = jnp.maximum(h2c, 0.0).astype(jnp.bfloat16)
        part = jax.lax.dot_general(
            h2c, w3b_ref[:, pl.ds(g * HC, HC)], (((1,), (1,)), ((), ())),
            preferred_element_type=jnp.float32)
        @pl.when(g == 0)
        def _():
            out_ref[...] = part + b3_ref[0]
        @pl.when((g > 0) & (g < G - 1))
        def _():
            out_ref[...] += part
        @pl.when(g == G - 1)
        def _():
            out_ref[...] = (out_ref[...] + part) * pw_ref[0]

    @pl.when((i >= nact) & (g == G - 1))
    def _():
        out_ref[...] = jnp.zeros_like(out_ref)


def _gffn(eot, meta, xg, W1, b1, W2, b2, W3, b3, pw):
    b1r = b1.reshape(E, 1, H)
    b2r = b2.reshape(E, 1, H)
    b3r = b3.reshape(E, 1, O)
    pw3 = pw.reshape(NTT, TN, 1)
    grid_spec = pltpu.PrefetchScalarGridSpec(
        num_scalar_prefetch=2,
        grid=(NTT, G),
        in_specs=[
            pl.BlockSpec((TN, D), lambda i, g, eot, meta: (i, 0)),
            pl.BlockSpec((1, H, D), lambda i, g, eot, meta: (eot[i], 0, 0)),
            pl.BlockSpec((1, 1, H), lambda i, g, eot, meta: (eot[i], 0, 0)),
            pl.BlockSpec((1, HC, H), lambda i, g, eot, meta: (eot[i], g, 0)),
            pl.BlockSpec((1, 1, HC), lambda i, g, eot, meta: (eot[i], 0, g)),
            pl.BlockSpec((1, O, HC), lambda i, g, eot, meta: (eot[i], 0, g)),
            pl.BlockSpec((1, 1, O), lambda i, g, eot, meta: (eot[i], 0, 0)),
            pl.BlockSpec((1, TN, 1), lambda i, g, eot, meta: (i, 0, 0)),
        ],
        out_specs=pl.BlockSpec((TN, O), lambda i, g, eot, meta: (i, 0)),
        scratch_shapes=[
            pltpu.VMEM((TN, H), jnp.bfloat16),
            pltpu.VMEM((H, D), jnp.bfloat16),
            pltpu.VMEM((H, H), jnp.bfloat16),
            pltpu.VMEM((O, H), jnp.bfloat16),
        ],
    )
    return pl.pallas_call(
        _gffn_body,
        grid_spec=grid_spec,
        out_shape=jax.ShapeDtypeStruct((P, O), jnp.float32),
    )(eot, meta, xg, W1, b1r, W2, b2r, W3, b3r, pw3)


def _combine_body(ys_hbm, pos2_hbm, out_hbm, p0_v, p1_v, r0_v, r1_v, sem0,
                  sem1):
    wid = lax.axis_index("s") * 2 + lax.axis_index("c")
    base_t = wid * _CT
    for ch in range(_CT // _CCH):
        t0 = base_t + ch * _CCH
        pltpu.sync_copy(pos2_hbm.at[pl.ds(t0, _CCH)], p0_v)
        pltpu.sync_copy(pos2_hbm.at[pl.ds(N + t0, _CCH)], p1_v)
        c0 = pltpu.async_copy(ys_hbm.at[p0_v], r0_v, sem0)
        c1 = pltpu.async_copy(ys_hbm.at[p1_v], r1_v, sem1)
        c0.wait()
        c1.wait()

        def body(r, _):
            for c in range(O // 16):
                sl = pl.ds(c * 16, 16)
                r0_v[r, sl] += r1_v[r, sl]
            return 0

        lax.fori_loop(0, _CCH, body, 0)
        pltpu.sync_copy(r0_v, out_hbm.at[pl.ds(t0, _CCH)])


def _combine(ys, pos2):
    mesh = plsc.VectorSubcoreMesh(core_axis_name="c", subcore_axis_name="s",
                                  num_cores=2, num_subcores=16)
    f = functools.partial(
        pl.kernel,
        out_type=jax.ShapeDtypeStruct((N, O), jnp.float32),
        mesh=mesh,
        scratch_types=[
            pltpu.VMEM((_CCH,), jnp.int32),
            pltpu.VMEM((_CCH,), jnp.int32),
            pltpu.VMEM((_CCH, O), jnp.float32),
            pltpu.VMEM((_CCH, O), jnp.float32),
            pltpu.SemaphoreType.DMA,
            pltpu.SemaphoreType.DMA,
        ],
    )
    return f(_combine_body)(ys, pos2)


def kernel(x, Wg, bg, W1, b1, W2, b2, W3, b3):
    probs_p, idx_p, wn_p, loss2 = _gating(x, Wg, bg)
    gate_probs = probs_p[:, :E]
    loss = loss2.reshape(())
    es = idx_p[:, :TOPK].reshape(S)
    ws = wn_p[:, :TOPK].reshape(S)

    cnt = _count(es)
    pw, pos2, eot, meta, xg = _assign(es, ws, cnt, x)
    ys = _gffn(eot, meta, xg, W1, b1, W2, b2, W3, b3, pw)
    final = _combine(ys, pos2)
    return (final, loss, gate_probs)


# speedup vs baseline: 1.1007x; 1.1007x over previous
import functools

import jax
import jax.numpy as jnp
from jax import lax
from jax.experimental import pallas as pl
from jax.experimental.pallas import tpu as pltpu
from jax.experimental.pallas import tpu_sc as plsc

N, D, H, O, E, TOPK = 2048, 1024, 2048, 1024, 8, 2
S = N * TOPK
TN = 256
P = S + E * TN
NTT = P // TN
G = 4
HC = H // G
EP = 128
TG = 256
NTG = N // TG

NW = 32
_RC = S // NW
_RCH = 32
_RNC = _RC // _RCH
_CT = N // NW
_CCH = 32


def _gating_body(x_ref, wg_ref, bg_ref, probs_ref, idx_ref, wn_ref, loss_ref,
                 acc_ref):
    i = pl.program_id(0)
    xt = x_ref[...]
    logits = jax.lax.dot_general(
        xt, wg_ref[...], (((1,), (0,)), ((), ())),
        preferred_element_type=jnp.float32) + bg_ref[...]
    col = jax.lax.broadcasted_iota(jnp.int32, (TG, EP), 1)
    valid = col < E
    logits = jnp.where(valid, logits, -jnp.inf)
    m = jnp.max(logits, axis=1, keepdims=True)
    ex = jnp.exp(logits - m)
    s = jnp.sum(ex, axis=1, keepdims=True)
    probs = ex / s
    probs_ref[...] = probs

    p1 = jnp.max(probs, axis=1, keepdims=True)
    i1 = jnp.min(jnp.where((probs == p1) & valid, col, EP), axis=1,
                 keepdims=True)
    one1 = col == i1
    probs_m = jnp.where(one1, -1.0, probs)
    p2 = jnp.max(probs_m, axis=1, keepdims=True)
    i2 = jnp.min(jnp.where((probs_m == p2) & valid, col, EP), axis=1,
                 keepdims=True)
    denom = p1 + p2
    idx_ref[...] = jnp.where(col == 0, i1, jnp.where(col == 1, i2, 0))
    wn_ref[...] = jnp.where(col == 0, p1 / denom,
                            jnp.where(col == 1, p2 / denom, 0.0))

    part = jnp.sum(probs, axis=0, keepdims=True)
    @pl.when(i == 0)
    def _():
        acc_ref[...] = part
    @pl.when(i > 0)
    def _():
        acc_ref[...] += part
    @pl.when(i == NTG - 1)
    def _():
        usage = acc_ref[...] / N
        lane = jax.lax.broadcasted_iota(jnp.int32, (1, EP), 1)
        uni = jnp.float32(1.0 / E)
        term = uni * (jnp.log(uni) - jnp.log(usage + 1e-8))
        loss_ref[...] = jnp.sum(jnp.where(lane < E, term, 0.0), axis=1,
                                keepdims=True) * 0.01


def _gating(x, Wg, bg):
    wgp = jnp.zeros((D, EP), jnp.float32).at[:, :E].set(Wg.T)
    bgp = jnp.zeros((1, EP), jnp.float32).at[0, :E].set(bg)
    return pl.pallas_call(
        _gating_body,
        grid=(NTG,),
        in_specs=[
            pl.BlockSpec((TG, D), lambda i: (i, 0)),
            pl.BlockSpec((D, EP), lambda i: (0, 0)),
            pl.BlockSpec((1, EP), lambda i: (0, 0)),
        ],
        out_specs=[
            pl.BlockSpec((TG, EP), lambda i: (i, 0)),
            pl.BlockSpec((TG, EP), lambda i: (i, 0)),
            pl.BlockSpec((TG, EP), lambda i: (i, 0)),
            pl.BlockSpec((1, 1), lambda i: (0, 0)),
        ],
        out_shape=[
            jax.ShapeDtypeStruct((N, EP), jnp.float32),
            jax.ShapeDtypeStruct((N, EP), jnp.int32),
            jax.ShapeDtypeStruct((N, EP), jnp.float32),
            jax.ShapeDtypeStruct((1, 1), jnp.float32),
        ],
        scratch_shapes=[pltpu.VMEM((1, EP), jnp.float32)],
    )(x, wgp, bgp)


def _count_body(es_hbm, cnt_hbm, es_v, cnt_v):
    wid = lax.axis_index("s") * 2 + lax.axis_index("c")
    base = wid * _RC
    lane = lax.iota(jnp.int32, 16)
    pltpu.sync_copy(es_hbm.at[pl.ds(base, _RC)], es_v)
    counts = [jnp.int32(0)] * E
    for v in range(_RC // 16):
        ids = es_v[pl.ds(v * 16, 16)]
        for e in range(E):
            counts[e] = counts[e] + jnp.sum(
                jnp.where(ids == e, 1, 0).astype(jnp.int32))
    cvec = jnp.zeros((16,), jnp.int32)
    for e in range(E):
        cvec = jnp.where(lane == e, counts[e], cvec)
    cnt_v[...] = cvec
    pltpu.sync_copy(cnt_v, cnt_hbm.at[wid])


def _count(es):
    mesh = plsc.VectorSubcoreMesh(core_axis_name="c", subcore_axis_name="s",
                                  num_cores=2, num_subcores=16)
    f = functools.partial(
        pl.kernel,
        out_type=jax.ShapeDtypeStruct((NW, 16), jnp.int32),
        mesh=mesh,
        scratch_types=[
            pltpu.VMEM((_RC,), jnp.int32),
            pltpu.VMEM((16,), jnp.int32),
        ],
        compiler_params=pltpu.CompilerParams(needs_layout_passes=False),
    )
    return f(_count_body)(es)


def _assign_body(es_hbm, ws_hbm, cnt_hbm, x_hbm, pw_hbm, pos2_hbm,
                 bt_hbm, ntl_hbm, xg_hbm, es_v, ws_v, posb, tokb, idxb,
                 allc_v, btb, ntlb, rb0, rb1, semg0, semg1, semw0, semw1,
                 sems):
    wid = lax.axis_index("s") * 2 + lax.axis_index("c")
    base = wid * _RC
    lane = lax.iota(jnp.int32, 16)
    zi = jnp.zeros((16,), jnp.int32)

    pltpu.sync_copy(es_hbm.at[pl.ds(base, _RC)], es_v)
    for c in range(_RNC):
        pltpu.sync_copy(ws_hbm.at[pl.ds(base + c * _RCH, _RCH)], ws_v.at[c])
    pltpu.sync_copy(cnt_hbm, allc_v)

    tvec = zi
    pvec = zi
    for w2 in range(NW):
        row = allc_v[w2]
        tvec = tvec + row
        pvec = pvec + jnp.where(jnp.int32(w2) < wid, row, zi)
    tot = [tvec[e] for e in range(E)]
    pref = [pvec[e] for e in range(E)]
    ntile = [(tot[e] + TN - 1) // TN for e in range(E)]
    base_tile = []
    bt = jnp.int32(0)
    for e in range(E):
        base_tile.append(bt)
        bt = bt + ntile[e]
    ntt_total = bt
    start = [base_tile[e] * TN + pref[e] for e in range(E)]

    for v in range(_RC // 16):
        ids = es_v[pl.ds(v * 16, 16)]
        pos = zi
        for e in range(E):
            msk = ids == e
            ones = jnp.where(msk, 1, 0).astype(jnp.int32)
            incl = plsc.cumsum(ones)
            pos = jnp.where(msk, start[e] + incl - 1, pos)
            start[e] = start[e] + jnp.sum(ones)
        slot = base + v * 16 + lane
        c, r = v // 2, (v % 2) * 16
        posb[c, pl.ds(r, 16)] = pos
        tokb[c, pl.ds(r, 16)] = slot >> 1
        idxb[c, pl.ds(r, 16)] = (slot & 1) * N + (slot >> 1)

    small = []
    for c in range(_RNC):
        small.append(pltpu.async_copy(ws_v.at[c], pw_hbm.at[posb.at[c]],
                                      sems))
        small.append(pltpu.async_copy(posb.at[c], pos2_hbm.at[idxb.at[c]],
                                      sems))

    rbufs = (rb0, rb1)
    gsems = (semg0, semg1)
    wsems = (semw0, semw1)
    g = {}
    w = {}
    g[0] = pltpu.async_copy(x_hbm.at[tokb.at[0]], rb0, semg0)
    g[1] = pltpu.async_copy(x_hbm.at[tokb.at[1]], rb1, semg1)
    for c in range(_RNC):
        g[c].wait()
        w[c] = pltpu.async_copy(rbufs[c % 2], xg_hbm.at[posb.at[c]],
                                wsems[c % 2])
        if c + 2 < _RNC:
            w[c].wait()
            g[c + 2] = pltpu.async_copy(x_hbm.at[tokb.at[c + 2]],
                                        rbufs[c % 2], gsems[c % 2])
    for c in range(max(0, _RNC - 2), _RNC):
        w[c].wait()
    for cp in small:
        cp.wait()

    @pl.when(wid == 0)
    def _():
        btv = zi
        ntlv = zi
        for e in range(E):
            btv = jnp.where(lane == e, base_tile[e], btv)
            ntlv = jnp.where(lane == e, ntile[e], ntlv)
        btb[...] = btv
        ntlb[...] = ntlv
        pltpu.sync_copy(btb, bt_hbm)
        pltpu.sync_copy(ntlb, ntl_hbm)


def _assign(es, ws, cnt, x):
    mesh = plsc.VectorSubcoreMesh(core_axis_name="c", subcore_axis_name="s",
                                  num_cores=2, num_subcores=16)
    f = functools.partial(
        pl.kernel,
        out_type=[
            jax.ShapeDtypeStruct((P,), jnp.float32),
            jax.ShapeDtypeStruct((2 * N,), jnp.int32),
            jax.ShapeDtypeStruct((16,), jnp.int32),
            jax.ShapeDtypeStruct((16,), jnp.int32),
            jax.ShapeDtypeStruct((P, D), jnp.float32),
        ],
        mesh=mesh,
        scratch_types=[
            pltpu.VMEM((_RC,), jnp.int32),
            pltpu.VMEM((_RNC, _RCH), jnp.float32),
            pltpu.VMEM((_RNC, _RCH), jnp.int32),
            pltpu.VMEM((_RNC, _RCH), jnp.int32),
            pltpu.VMEM((_RNC, _RCH), jnp.int32),
            pltpu.VMEM((NW, 16), jnp.int32),
            pltpu.VMEM((16,), jnp.int32),
            pltpu.VMEM((16,), jnp.int32),
            pltpu.VMEM((_RCH, D), jnp.float32),
            pltpu.VMEM((_RCH, D), jnp.float32),
            pltpu.SemaphoreType.DMA,
            pltpu.SemaphoreType.DMA,
            pltpu.SemaphoreType.DMA,
            pltpu.SemaphoreType.DMA,
            pltpu.SemaphoreType.DMA,
        ],
        compiler_params=pltpu.CompilerParams(needs_layout_passes=False),
    )
    return f(_assign_body)(es, ws, cnt, x)


TMAX = N // TN


def _gffn_body(bt_ref, ntl_ref, xg_ref, w1_ref, b1_ref, w2_ref, b2_ref,
               w3_ref, b3_ref, pw_ref, out_ref, h1_ref, w1b_ref, w2b_ref,
               w3b_ref):
    e = pl.program_id(0)
    t = pl.program_id(1)
    g = pl.program_id(2)
    ntile_e = ntl_ref[e]

    @pl.when((t == 0) & (ntile_e > 0))
    def _():
        w2b_ref[pl.ds(g * HC, HC), :] = w2_ref[0].astype(jnp.bfloat16)
        w3b_ref[:, pl.ds(g * HC, HC)] = w3_ref[0].astype(jnp.bfloat16)
        @pl.when(g == 0)
        def _():
            w1b_ref[...] = w1_ref[0].astype(jnp.bfloat16)

    @pl.when(t < ntile_e)
    def _():
        @pl.when(g == 0)
        def _():
            h1 = jax.lax.dot_general(
                xg_ref[...].astype(jnp.bfloat16), w1b_ref[...],
                (((1,), (1,)), ((), ())),
                preferred_element_type=jnp.float32) + b1_ref[0]
            h1_ref[...] = jnp.maximum(h1, 0.0).astype(jnp.bfloat16)
        h2c = jax.lax.dot_general(
            h1_ref[...], w2b_ref[pl.ds(g * HC, HC), :],
            (((1,), (1,)), ((), ())),
            preferred_element_type=jnp.float32) + b2_ref[0, :, pl.ds(g * HC, HC)]
        h2c = jnp.maximum(h2c, 0.0).astype(jnp.bfloat16)
        part = jax.lax.dot_general(
            h2c, w3b_ref[:, pl.ds(g * HC, HC)], (((1,), (1,)), ((), ())),
            preferred_element_type=jnp.float32)
        @pl.when(g == 0)
        def _():
            out_ref[...] = part + b3_ref[0]
        @pl.when((g > 0) & (g < G - 1))
        def _():
            out_ref[...] += part
        @pl.when(g == G - 1)
        def _():
            out_ref[...] = (out_ref[...] + part) * pw_ref[0]


def _tix(t, e, bt, ntl):
    return jnp.minimum(bt[e] + jnp.minimum(t, jnp.maximum(ntl[e] - 1, 0)),
                       NTT - 1)


def _gffn(bt, ntl, xg, W1, b1, W2, b2, W3, b3, pw):
    b1r = b1.reshape(E, 1, H)
    b2r = b2.reshape(E, 1, H)
    b3r = b3.reshape(E, 1, O)
    pw3 = pw.reshape(NTT, TN, 1)
    grid_spec = pltpu.PrefetchScalarGridSpec(
        num_scalar_prefetch=2,
        grid=(E, TMAX, G),
        in_specs=[
            pl.BlockSpec((TN, D),
                         lambda e, t, g, bt, ntl: (_tix(t, e, bt, ntl), 0)),
            pl.BlockSpec((1, H, D), lambda e, t, g, bt, ntl: (e, 0, 0)),
            pl.BlockSpec((1, 1, H), lambda e, t, g, bt, ntl: (e, 0, 0)),
            pl.BlockSpec((1, HC, H),
                         lambda e, t, g, bt, ntl:
                         (e, jnp.where(t == 0, g, G - 1), 0)),
            pl.BlockSpec((1, 1, H), lambda e, t, g, bt, ntl: (e, 0, 0)),
            pl.BlockSpec((1, O, HC),
                         lambda e, t, g, bt, ntl:
                         (e, 0, jnp.where(t == 0, g, G - 1))),
            pl.BlockSpec((1, 1, O), lambda e, t, g, bt, ntl: (e, 0, 0)),
            pl.BlockSpec((1, TN, 1),
                         lambda e, t, g, bt, ntl:
                         (_tix(t, e, bt, ntl), 0, 0)),
        ],
        out_specs=pl.BlockSpec((TN, O),
                               lambda e, t, g, bt, ntl:
                               (_tix(t, e, bt, ntl), 0)),
        scratch_shapes=[
            pltpu.VMEM((TN, H), jnp.bfloat16),
            pltpu.VMEM((H, D), jnp.bfloat16),
            pltpu.VMEM((H, H), jnp.bfloat16),
            pltpu.VMEM((O, H), jnp.bfloat16),
        ],
    )
    return pl.pallas_call(
        _gffn_body,
        grid_spec=grid_spec,
        out_shape=jax.ShapeDtypeStruct((P, O), jnp.float32),
    )(bt, ntl, xg, W1, b1r, W2, b2r, W3, b3r, pw3)


def _combine_body(ys_hbm, pos2_hbm, out_hbm, p0_v, p1_v, r0_v, r1_v, sem0,
                  sem1):
    wid = lax.axis_index("s") * 2 + lax.axis_index("c")
    base_t = wid * _CT
    for ch in range(_CT // _CCH):
        t0 = base_t + ch * _CCH
        pltpu.sync_copy(pos2_hbm.at[pl.ds(t0, _CCH)], p0_v)
        pltpu.sync_copy(pos2_hbm.at[pl.ds(N + t0, _CCH)], p1_v)
        c0 = pltpu.async_copy(ys_hbm.at[p0_v], r0_v, sem0)
        c1 = pltpu.async_copy(ys_hbm.at[p1_v], r1_v, sem1)
        c0.wait()
        c1.wait()

        def body(r, _):
            for c in range(O // 16):
                sl = pl.ds(c * 16, 16)
                r0_v[r, sl] += r1_v[r, sl]
            return 0

        lax.fori_loop(0, _CCH, body, 0)
        pltpu.sync_copy(r0_v, out_hbm.at[pl.ds(t0, _CCH)])


def _combine(ys, pos2):
    mesh = plsc.VectorSubcoreMesh(core_axis_name="c", subcore_axis_name="s",
                                  num_cores=2, num_subcores=16)
    f = functools.partial(
        pl.kernel,
        out_type=jax.ShapeDtypeStruct((N, O), jnp.float32),
        mesh=mesh,
        scratch_types=[
            pltpu.VMEM((_CCH,), jnp.int32),
            pltpu.VMEM((_CCH,), jnp.int32),
            pltpu.VMEM((_CCH, O), jnp.float32),
            pltpu.VMEM((_CCH, O), jnp.float32),
            pltpu.SemaphoreType.DMA,
            pltpu.SemaphoreType.DMA,
        ],
    )
    return f(_combine_body)(ys, pos2)


def kernel(x, Wg, bg, W1, b1, W2, b2, W3, b3):
    probs_p, idx_p, wn_p, loss2 = _gating(x, Wg, bg)
    gate_probs = probs_p[:, :E]
    loss = loss2.reshape(())
    es = idx_p[:, :TOPK].reshape(S)
    ws = wn_p[:, :TOPK].reshape(S)

    cnt = _count(es)
    pw, pos2, bt, ntl, xg = _assign(es, ws, cnt, x)
    ys = _gffn(bt, ntl, xg, W1, b1, W2, b2, W3, b3, pw)
    final = _combine(ys, pos2)
    return (final, loss, gate_probs)


# speedup vs baseline: 1.1143x; 1.0123x over previous
import functools

import jax
import jax.numpy as jnp
from jax import lax
from jax.experimental import pallas as pl
from jax.experimental.pallas import tpu as pltpu
from jax.experimental.pallas import tpu_sc as plsc

N, D, H, O, E, TOPK = 2048, 1024, 2048, 1024, 8, 2
S = N * TOPK
TN = 256
P = S + E * TN
NTT = P // TN
G = 4
HC = H // G
EP = 128
TG = 256
NTG = N // TG

NW = 32
_RC = S // NW
_RCH = 32
_RNC = _RC // _RCH
_CT = N // NW
_CCH = 16


def _gating_body(x_ref, wg_ref, bg_ref, probs_ref, idx_ref, wn_ref, loss_ref,
                 acc_ref):
    i = pl.program_id(0)
    xt = x_ref[...]
    logits = jax.lax.dot_general(
        xt, wg_ref[...], (((1,), (0,)), ((), ())),
        preferred_element_type=jnp.float32) + bg_ref[...]
    col = jax.lax.broadcasted_iota(jnp.int32, (TG, EP), 1)
    valid = col < E
    logits = jnp.where(valid, logits, -jnp.inf)
    m = jnp.max(logits, axis=1, keepdims=True)
    ex = jnp.exp(logits - m)
    s = jnp.sum(ex, axis=1, keepdims=True)
    probs = ex / s
    probs_ref[...] = probs

    p1 = jnp.max(probs, axis=1, keepdims=True)
    i1 = jnp.min(jnp.where((probs == p1) & valid, col, EP), axis=1,
                 keepdims=True)
    one1 = col == i1
    probs_m = jnp.where(one1, -1.0, probs)
    p2 = jnp.max(probs_m, axis=1, keepdims=True)
    i2 = jnp.min(jnp.where((probs_m == p2) & valid, col, EP), axis=1,
                 keepdims=True)
    denom = p1 + p2
    idx_ref[...] = jnp.where(col == 0, i1, jnp.where(col == 1, i2, 0))
    wn_ref[...] = jnp.where(col == 0, p1 / denom,
                            jnp.where(col == 1, p2 / denom, 0.0))

    part = jnp.sum(probs, axis=0, keepdims=True)
    @pl.when(i == 0)
    def _():
        acc_ref[...] = part
    @pl.when(i > 0)
    def _():
        acc_ref[...] += part
    @pl.when(i == NTG - 1)
    def _():
        usage = acc_ref[...] / N
        lane = jax.lax.broadcasted_iota(jnp.int32, (1, EP), 1)
        uni = jnp.float32(1.0 / E)
        term = uni * (jnp.log(uni) - jnp.log(usage + 1e-8))
        loss_ref[...] = jnp.sum(jnp.where(lane < E, term, 0.0), axis=1,
                                keepdims=True) * 0.01


def _gating(x, Wg, bg):
    wgp = jnp.zeros((D, EP), jnp.float32).at[:, :E].set(Wg.T)
    bgp = jnp.zeros((1, EP), jnp.float32).at[0, :E].set(bg)
    return pl.pallas_call(
        _gating_body,
        grid=(NTG,),
        in_specs=[
            pl.BlockSpec((TG, D), lambda i: (i, 0)),
            pl.BlockSpec((D, EP), lambda i: (0, 0)),
            pl.BlockSpec((1, EP), lambda i: (0, 0)),
        ],
        out_specs=[
            pl.BlockSpec((TG, EP), lambda i: (i, 0)),
            pl.BlockSpec((TG, EP), lambda i: (i, 0)),
            pl.BlockSpec((TG, EP), lambda i: (i, 0)),
            pl.BlockSpec((1, 1), lambda i: (0, 0)),
        ],
        out_shape=[
            jax.ShapeDtypeStruct((N, EP), jnp.float32),
            jax.ShapeDtypeStruct((N, EP), jnp.int32),
            jax.ShapeDtypeStruct((N, EP), jnp.float32),
            jax.ShapeDtypeStruct((1, 1), jnp.float32),
        ],
        scratch_shapes=[pltpu.VMEM((1, EP), jnp.float32)],
    )(x, wgp, bgp)


def _count_body(es_hbm, cnt_hbm, es_v, cnt_v):
    wid = lax.axis_index("s") * 2 + lax.axis_index("c")
    base = wid * _RC
    lane = lax.iota(jnp.int32, 16)
    pltpu.sync_copy(es_hbm.at[pl.ds(base, _RC)], es_v)
    counts = [jnp.int32(0)] * E
    for v in range(_RC // 16):
        ids = es_v[pl.ds(v * 16, 16)]
        for e in range(E):
            counts[e] = counts[e] + jnp.sum(
                jnp.where(ids == e, 1, 0).astype(jnp.int32))
    cvec = jnp.zeros((16,), jnp.int32)
    for e in range(E):
        cvec = jnp.where(lane == e, counts[e], cvec)
    cnt_v[...] = cvec
    pltpu.sync_copy(cnt_v, cnt_hbm.at[wid])


def _count(es):
    mesh = plsc.VectorSubcoreMesh(core_axis_name="c", subcore_axis_name="s",
                                  num_cores=2, num_subcores=16)
    f = functools.partial(
        pl.kernel,
        out_type=jax.ShapeDtypeStruct((NW, 16), jnp.int32),
        mesh=mesh,
        scratch_types=[
            pltpu.VMEM((_RC,), jnp.int32),
            pltpu.VMEM((16,), jnp.int32),
        ],
        compiler_params=pltpu.CompilerParams(needs_layout_passes=False),
    )
    return f(_count_body)(es)


def _assign_body(es_hbm, ws_hbm, cnt_hbm, x_hbm, pw_hbm, pos2_hbm,
                 bt_hbm, ntl_hbm, xg_hbm, es_v, ws_v, posb, tokb, idxb,
                 allc_v, btb, ntlb, rb0, rb1, rb2, semg0, semg1, semg2,
                 semw0, semw1, semw2, sems):
    wid = lax.axis_index("s") * 2 + lax.axis_index("c")
    base = wid * _RC
    lane = lax.iota(jnp.int32, 16)
    zi = jnp.zeros((16,), jnp.int32)

    pltpu.sync_copy(es_hbm.at[pl.ds(base, _RC)], es_v)
    for c in range(_RNC):
        pltpu.sync_copy(ws_hbm.at[pl.ds(base + c * _RCH, _RCH)], ws_v.at[c])
    pltpu.sync_copy(cnt_hbm, allc_v)

    tvec = zi
    pvec = zi
    for w2 in range(NW):
        row = allc_v[w2]
        tvec = tvec + row
        pvec = pvec + jnp.where(jnp.int32(w2) < wid, row, zi)
    tot = [tvec[e] for e in range(E)]
    pref = [pvec[e] for e in range(E)]
    ntile = [(tot[e] + TN - 1) // TN for e in range(E)]
    base_tile = []
    bt = jnp.int32(0)
    for e in range(E):
        base_tile.append(bt)
        bt = bt + ntile[e]
    ntt_total = bt
    start = [base_tile[e] * TN + pref[e] for e in range(E)]

    for v in range(_RC // 16):
        ids = es_v[pl.ds(v * 16, 16)]
        pos = zi
        for e in range(E):
            msk = ids == e
            ones = jnp.where(msk, 1, 0).astype(jnp.int32)
            incl = plsc.cumsum(ones)
            pos = jnp.where(msk, start[e] + incl - 1, pos)
            start[e] = start[e] + jnp.sum(ones)
        slot = base + v * 16 + lane
        c, r = v // 2, (v % 2) * 16
        posb[c, pl.ds(r, 16)] = pos
        tokb[c, pl.ds(r, 16)] = slot >> 1
        idxb[c, pl.ds(r, 16)] = (slot & 1) * N + (slot >> 1)

    small = []
    for c in range(_RNC):
        small.append(pltpu.async_copy(ws_v.at[c], pw_hbm.at[posb.at[c]],
                                      sems))
        small.append(pltpu.async_copy(posb.at[c], pos2_hbm.at[idxb.at[c]],
                                      sems))

    rbufs = (rb0, rb1, rb2)
    gsems = (semg0, semg1, semg2)
    wsems = (semw0, semw1, semw2)
    g = {}
    w = {}
    for c in range(3):
        g[c] = pltpu.async_copy(x_hbm.at[tokb.at[c]], rbufs[c], gsems[c])
    for c in range(_RNC):
        g[c].wait()
        w[c] = pltpu.async_copy(rbufs[c % 3], xg_hbm.at[posb.at[c]],
                                wsems[c % 3])
        if c + 3 < _RNC:
            w[c].wait()
            g[c + 3] = pltpu.async_copy(x_hbm.at[tokb.at[c + 3]],
                                        rbufs[c % 3], gsems[c % 3])
    for c in range(max(0, _RNC - 3), _RNC):
        w[c].wait()
    for cp in small:
        cp.wait()

    @pl.when(wid == 0)
    def _():
        btv = zi
        ntlv = zi
        for e in range(E):
            btv = jnp.where(lane == e, base_tile[e], btv)
            ntlv = jnp.where(lane == e, ntile[e], ntlv)
        btb[...] = btv
        ntlb[...] = ntlv
        pltpu.sync_copy(btb, bt_hbm)
        pltpu.sync_copy(ntlb, ntl_hbm)


def _assign(es, ws, cnt, x):
    mesh = plsc.VectorSubcoreMesh(core_axis_name="c", subcore_axis_name="s",
                                  num_cores=2, num_subcores=16)
    f = functools.partial(
        pl.kernel,
        out_type=[
            jax.ShapeDtypeStruct((P,), jnp.float32),
            jax.ShapeDtypeStruct((2 * N,), jnp.int32),
            jax.ShapeDtypeStruct((16,), jnp.int32),
            jax.ShapeDtypeStruct((16,), jnp.int32),
            jax.ShapeDtypeStruct((P, D), jnp.float32),
        ],
        mesh=mesh,
        scratch_types=[
            pltpu.VMEM((_RC,), jnp.int32),
            pltpu.VMEM((_RNC, _RCH), jnp.float32),
            pltpu.VMEM((_RNC, _RCH), jnp.int32),
            pltpu.VMEM((_RNC, _RCH), jnp.int32),
            pltpu.VMEM((_RNC, _RCH), jnp.int32),
            pltpu.VMEM((NW, 16), jnp.int32),
            pltpu.VMEM((16,), jnp.int32),
            pltpu.VMEM((16,), jnp.int32),
            pltpu.VMEM((_RCH, D), jnp.float32),
            pltpu.VMEM((_RCH, D), jnp.float32),
            pltpu.VMEM((_RCH, D), jnp.float32),
            pltpu.SemaphoreType.DMA,
            pltpu.SemaphoreType.DMA,
            pltpu.SemaphoreType.DMA,
            pltpu.SemaphoreType.DMA,
            pltpu.SemaphoreType.DMA,
            pltpu.SemaphoreType.DMA,
            pltpu.SemaphoreType.DMA,
        ],
        compiler_params=pltpu.CompilerParams(needs_layout_passes=False),
    )
    return f(_assign_body)(es, ws, cnt, x)


TMAX = N // TN


def _gffn_body(bt_ref, ntl_ref, xg_ref, w1_ref, b1_ref, w2_ref, b2_ref,
               w3_ref, b3_ref, pw_ref, out_ref, h1_ref, w1b_ref, w2b_ref,
               w3b_ref):
    e = pl.program_id(0)
    t = pl.program_id(1)
    g = pl.program_id(2)
    ntile_e = ntl_ref[e]

    @pl.when((t == 0) & (ntile_e > 0))
    def _():
        w2b_ref[pl.ds(g * HC, HC), :] = w2_ref[0].astype(jnp.bfloat16)
        w3b_ref[:, pl.ds(g * HC, HC)] = w3_ref[0].astype(jnp.bfloat16)
        @pl.when(g == 0)
        def _():
            w1b_ref[...] = w1_ref[0].astype(jnp.bfloat16)

    @pl.when(t < ntile_e)
    def _():
        @pl.when(g == 0)
        def _():
            h1 = jax.lax.dot_general(
                xg_ref[...].astype(jnp.bfloat16), w1b_ref[...],
                (((1,), (1,)), ((), ())),
                preferred_element_type=jnp.float32) + b1_ref[0]
            h1_ref[...] = jnp.maximum(h1, 0.0).astype(jnp.bfloat16)
        h2c = jax.lax.dot_general(
            h1_ref[...], w2b_ref[pl.ds(g * HC, HC), :],
            (((1,), (1,)), ((), ())),
            preferred_element_type=jnp.float32) + b2_ref[0, :, pl.ds(g * HC, HC)]
        h2c = jnp.maximum(h2c, 0.0).astype(jnp.bfloat16)
        part = jax.lax.dot_general(
            h2c, w3b_ref[:, pl.ds(g * HC, HC)], (((1,), (1,)), ((), ())),
            preferred_element_type=jnp.float32)
        @pl.when(g == 0)
        def _():
            out_ref[...] = part + b3_ref[0]
        @pl.when((g > 0) & (g < G - 1))
        def _():
            out_ref[...] += part
        @pl.when(g == G - 1)
        def _():
            out_ref[...] = (out_ref[...] + part) * pw_ref[0]


def _tix(t, e, bt, ntl):
    return jnp.minimum(bt[e] + jnp.minimum(t, jnp.maximum(ntl[e] - 1, 0)),
                       NTT - 1)


def _gffn(bt, ntl, xg, W1, b1, W2, b2, W3, b3, pw):
    b1r = b1.reshape(E, 1, H)
    b2r = b2.reshape(E, 1, H)
    b3r = b3.reshape(E, 1, O)
    pw3 = pw.reshape(NTT, TN, 1)
    grid_spec = pltpu.PrefetchScalarGridSpec(
        num_scalar_prefetch=2,
        grid=(E, TMAX, G),
        in_specs=[
            pl.BlockSpec((TN, D),
                         lambda e, t, g, bt, ntl: (_tix(t, e, bt, ntl), 0)),
            pl.BlockSpec((1, H, D), lambda e, t, g, bt, ntl: (e, 0, 0)),
            pl.BlockSpec((1, 1, H), lambda e, t, g, bt, ntl: (e, 0, 0)),
            pl.BlockSpec((1, HC, H),
                         lambda e, t, g, bt, ntl:
                         (e, jnp.where(t == 0, g, G - 1), 0)),
            pl.BlockSpec((1, 1, H), lambda e, t, g, bt, ntl: (e, 0, 0)),
            pl.BlockSpec((1, O, HC),
                         lambda e, t, g, bt, ntl:
                         (e, 0, jnp.where(t == 0, g, G - 1))),
            pl.BlockSpec((1, 1, O), lambda e, t, g, bt, ntl: (e, 0, 0)),
            pl.BlockSpec((1, TN, 1),
                         lambda e, t, g, bt, ntl:
                         (_tix(t, e, bt, ntl), 0, 0)),
        ],
        out_specs=pl.BlockSpec((TN, O),
                               lambda e, t, g, bt, ntl:
                               (_tix(t, e, bt, ntl), 0)),
        scratch_shapes=[
            pltpu.VMEM((TN, H), jnp.bfloat16),
            pltpu.VMEM((H, D), jnp.bfloat16),
            pltpu.VMEM((H, H), jnp.bfloat16),
            pltpu.VMEM((O, H), jnp.bfloat16),
        ],
    )
    return pl.pallas_call(
        _gffn_body,
        grid_spec=grid_spec,
        out_shape=jax.ShapeDtypeStruct((P, O), jnp.float32),
    )(bt, ntl, xg, W1, b1r, W2, b2r, W3, b3r, pw3)


def _combine_body(ys_hbm, pos2_hbm, out_hbm, p0_v, p1_v, r0a, r1a, r0b, r1b,
                  sga0, sga1, sgb0, sgb1, swa, swb):
    wid = lax.axis_index("s") * 2 + lax.axis_index("c")
    base_t = wid * _CT
    nch = _CT // _CCH
    pltpu.sync_copy(pos2_hbm.at[pl.ds(base_t, _CT)], p0_v)
    pltpu.sync_copy(pos2_hbm.at[pl.ds(N + base_t, _CT)], p1_v)
    r0s = (r0a, r0b)
    r1s = (r1a, r1b)
    gs0 = (sga0, sgb0)
    gs1 = (sga1, sgb1)
    ws = (swa, swb)

    def start_pair(c):
        sl = pl.ds(c * _CCH, _CCH)
        return (pltpu.async_copy(ys_hbm.at[p0_v.at[sl]], r0s[c % 2],
                                 gs0[c % 2]),
                pltpu.async_copy(ys_hbm.at[p1_v.at[sl]], r1s[c % 2],
                                 gs1[c % 2]))

    g = {0: start_pair(0), 1: start_pair(1)}
    w = {}
    for c in range(nch):
        g[c][0].wait()
        g[c][1].wait()
        r0_v = r0s[c % 2]
        r1_v = r1s[c % 2]

        def body(r, _):
            for cc in range(O // 16):
                sl = pl.ds(cc * 16, 16)
                r0_v[r, sl] += r1_v[r, sl]
            return 0

        lax.fori_loop(0, _CCH, body, 0)
        w[c] = pltpu.async_copy(r0_v, out_hbm.at[pl.ds(base_t + c * _CCH,
                                                       _CCH)], ws[c % 2])
        if c + 2 < nch:
            w[c].wait()
            g[c + 2] = start_pair(c + 2)
    for c in range(max(0, nch - 2), nch):
        w[c].wait()


def _combine(ys, pos2):
    mesh = plsc.VectorSubcoreMesh(core_axis_name="c", subcore_axis_name="s",
                                  num_cores=2, num_subcores=16)
    f = functools.partial(
        pl.kernel,
        out_type=jax.ShapeDtypeStruct((N, O), jnp.float32),
        mesh=mesh,
        scratch_types=[
            pltpu.VMEM((_CT,), jnp.int32),
            pltpu.VMEM((_CT,), jnp.int32),
            pltpu.VMEM((_CCH, O), jnp.float32),
            pltpu.VMEM((_CCH, O), jnp.float32),
            pltpu.VMEM((_CCH, O), jnp.float32),
            pltpu.VMEM((_CCH, O), jnp.float32),
            pltpu.SemaphoreType.DMA,
            pltpu.SemaphoreType.DMA,
            pltpu.SemaphoreType.DMA,
            pltpu.SemaphoreType.DMA,
            pltpu.SemaphoreType.DMA,
            pltpu.SemaphoreType.DMA,
        ],
    )
    return f(_combine_body)(ys, pos2)


def kernel(x, Wg, bg, W1, b1, W2, b2, W3, b3):
    probs_p, idx_p, wn_p, loss2 = _gating(x, Wg, bg)
    gate_probs = probs_p[:, :E]
    loss = loss2.reshape(())
    es = idx_p[:, :TOPK].reshape(S)
    ws = wn_p[:, :TOPK].reshape(S)

    cnt = _count(es)
    pw, pos2, bt, ntl, xg = _assign(es, ws, cnt, x)
    ys = _gffn(bt, ntl, xg, W1, b1, W2, b2, W3, b3, pw)
    final = _combine(ys, pos2)
    return (final, loss, gate_probs)


# speedup vs baseline: 1.2069x; 1.0832x over previous
import functools

import jax
import jax.numpy as jnp
from jax import lax
from jax.experimental import pallas as pl
from jax.experimental.pallas import tpu as pltpu
from jax.experimental.pallas import tpu_sc as plsc

N, D, H, O, E, TOPK = 2048, 1024, 2048, 1024, 8, 2
S = N * TOPK
TN = 256
P = S + E * TN
NTT = P // TN
G = 4
HC = H // G
EP = 128
TG = 256
NTG = N // TG

NW = 32
_RC = S // NW
_RCH = 32
_RNC = _RC // _RCH
_CT = N // NW
_CCH = 16


def _gating_body(x_ref, wg_ref, bg_ref, probs_ref, idx_ref, wn_ref, loss_ref,
                 acc_ref):
    i = pl.program_id(0)
    xt = x_ref[...]
    logits = jax.lax.dot_general(
        xt, wg_ref[...], (((1,), (0,)), ((), ())),
        preferred_element_type=jnp.float32) + bg_ref[...]
    col = jax.lax.broadcasted_iota(jnp.int32, (TG, EP), 1)
    valid = col < E
    logits = jnp.where(valid, logits, -jnp.inf)
    m = jnp.max(logits, axis=1, keepdims=True)
    ex = jnp.exp(logits - m)
    s = jnp.sum(ex, axis=1, keepdims=True)
    probs = ex / s
    probs_ref[...] = probs

    p1 = jnp.max(probs, axis=1, keepdims=True)
    i1 = jnp.min(jnp.where((probs == p1) & valid, col, EP), axis=1,
                 keepdims=True)
    one1 = col == i1
    probs_m = jnp.where(one1, -1.0, probs)
    p2 = jnp.max(probs_m, axis=1, keepdims=True)
    i2 = jnp.min(jnp.where((probs_m == p2) & valid, col, EP), axis=1,
                 keepdims=True)
    denom = p1 + p2
    idx_ref[...] = jnp.where(col == 0, i1, jnp.where(col == 1, i2, 0))
    wn_ref[...] = jnp.where(col == 0, p1 / denom,
                            jnp.where(col == 1, p2 / denom, 0.0))

    part = jnp.sum(probs, axis=0, keepdims=True)
    @pl.when(i == 0)
    def _():
        acc_ref[...] = part
    @pl.when(i > 0)
    def _():
        acc_ref[...] += part
    @pl.when(i == NTG - 1)
    def _():
        usage = acc_ref[...] / N
        lane = jax.lax.broadcasted_iota(jnp.int32, (1, EP), 1)
        uni = jnp.float32(1.0 / E)
        term = uni * (jnp.log(uni) - jnp.log(usage + 1e-8))
        loss_ref[...] = jnp.sum(jnp.where(lane < E, term, 0.0), axis=1,
                                keepdims=True) * 0.01


def _gating(x, Wg, bg):
    wgp = jnp.zeros((D, EP), jnp.float32).at[:, :E].set(Wg.T)
    bgp = jnp.zeros((1, EP), jnp.float32).at[0, :E].set(bg)
    return pl.pallas_call(
        _gating_body,
        grid=(NTG,),
        in_specs=[
            pl.BlockSpec((TG, D), lambda i: (i, 0)),
            pl.BlockSpec((D, EP), lambda i: (0, 0)),
            pl.BlockSpec((1, EP), lambda i: (0, 0)),
        ],
        out_specs=[
            pl.BlockSpec((TG, EP), lambda i: (i, 0)),
            pl.BlockSpec((TG, EP), lambda i: (i, 0)),
            pl.BlockSpec((TG, EP), lambda i: (i, 0)),
            pl.BlockSpec((1, 1), lambda i: (0, 0)),
        ],
        out_shape=[
            jax.ShapeDtypeStruct((N, EP), jnp.float32),
            jax.ShapeDtypeStruct((N, EP), jnp.int32),
            jax.ShapeDtypeStruct((N, EP), jnp.float32),
            jax.ShapeDtypeStruct((1, 1), jnp.float32),
        ],
        scratch_shapes=[pltpu.VMEM((1, EP), jnp.float32)],
    )(x, wgp, bgp)


def _count_body(es_hbm, cnt_hbm, es_v, cnt_v):
    wid = lax.axis_index("s") * 2 + lax.axis_index("c")
    base = wid * _RC
    lane = lax.iota(jnp.int32, 16)
    pltpu.sync_copy(es_hbm.at[pl.ds(base, _RC)], es_v)
    counts = [jnp.int32(0)] * E
    for v in range(_RC // 16):
        ids = es_v[pl.ds(v * 16, 16)]
        for e in range(E):
            counts[e] = counts[e] + jnp.sum(
                jnp.where(ids == e, 1, 0).astype(jnp.int32))
    cvec = jnp.zeros((16,), jnp.int32)
    for e in range(E):
        cvec = jnp.where(lane == e, counts[e], cvec)
    cnt_v[...] = cvec
    pltpu.sync_copy(cnt_v, cnt_hbm.at[wid])


def _count(es):
    mesh = plsc.VectorSubcoreMesh(core_axis_name="c", subcore_axis_name="s",
                                  num_cores=2, num_subcores=16)
    f = functools.partial(
        pl.kernel,
        out_type=jax.ShapeDtypeStruct((NW, 16), jnp.int32),
        mesh=mesh,
        scratch_types=[
            pltpu.VMEM((_RC,), jnp.int32),
            pltpu.VMEM((16,), jnp.int32),
        ],
        compiler_params=pltpu.CompilerParams(needs_layout_passes=False),
    )
    return f(_count_body)(es)


def _assign_body(es_hbm, ws_hbm, cnt_hbm, x_hbm, pw_hbm, pos2_hbm,
                 bt_hbm, se_hbm, sg_hbm, stx_hbm, sft_hbm, xg_hbm,
                 es_v, ws_v, posb, tokb, idxb, allc_v, btb, seb, sgb, stxb,
                 sftb, rb0, rb1, rb2, semg0, semg1, semg2,
                 semw0, semw1, semw2, sems):
    wid = lax.axis_index("s") * 2 + lax.axis_index("c")
    base = wid * _RC
    lane = lax.iota(jnp.int32, 16)
    zi = jnp.zeros((16,), jnp.int32)

    pltpu.sync_copy(es_hbm.at[pl.ds(base, _RC)], es_v)
    for c in range(_RNC):
        pltpu.sync_copy(ws_hbm.at[pl.ds(base + c * _RCH, _RCH)], ws_v.at[c])
    pltpu.sync_copy(cnt_hbm, allc_v)

    tvec = zi
    pvec = zi
    for w2 in range(NW):
        row = allc_v[w2]
        tvec = tvec + row
        pvec = pvec + jnp.where(jnp.int32(w2) < wid, row, zi)
    tot = [tvec[e] for e in range(E)]
    pref = [pvec[e] for e in range(E)]
    ntile = [(tot[e] + TN - 1) // TN for e in range(E)]
    base_tile = []
    bt = jnp.int32(0)
    for e in range(E):
        base_tile.append(bt)
        bt = bt + ntile[e]
    ntt_total = bt
    start = [base_tile[e] * TN + pref[e] for e in range(E)]

    for v in range(_RC // 16):
        ids = es_v[pl.ds(v * 16, 16)]
        pos = zi
        for e in range(E):
            msk = ids == e
            ones = jnp.where(msk, 1, 0).astype(jnp.int32)
            incl = plsc.cumsum(ones)
            pos = jnp.where(msk, start[e] + incl - 1, pos)
            start[e] = start[e] + jnp.sum(ones)
        slot = base + v * 16 + lane
        c, r = v // 2, (v % 2) * 16
        posb[c, pl.ds(r, 16)] = pos
        tokb[c, pl.ds(r, 16)] = slot >> 1
        idxb[c, pl.ds(r, 16)] = (slot & 1) * N + (slot >> 1)

    small = []
    for c in range(_RNC):
        small.append(pltpu.async_copy(ws_v.at[c], pw_hbm.at[posb.at[c]],
                                      sems))
        small.append(pltpu.async_copy(posb.at[c], pos2_hbm.at[idxb.at[c]],
                                      sems))

    rbufs = (rb0, rb1, rb2)
    gsems = (semg0, semg1, semg2)
    wsems = (semw0, semw1, semw2)
    g = {}
    w = {}
    for c in range(3):
        g[c] = pltpu.async_copy(x_hbm.at[tokb.at[c]], rbufs[c], gsems[c])
    for c in range(_RNC):
        g[c].wait()
        w[c] = pltpu.async_copy(rbufs[c % 3], xg_hbm.at[posb.at[c]],
                                wsems[c % 3])
        if c + 3 < _RNC:
            w[c].wait()
            g[c + 3] = pltpu.async_copy(x_hbm.at[tokb.at[c + 3]],
                                        rbufs[c % 3], gsems[c % 3])
    for c in range(max(0, _RNC - 3), _RNC):
        w[c].wait()
    for cp in small:
        cp.wait()

    @pl.when(wid == 0)
    def _():
        nsteps = ntt_total * G
        for ch in range(MAXS // 16):
            s = ch * 16 + lane
            sc = jnp.minimum(s, nsteps - 1)
            e_v = zi - 1
            for e in range(E):
                e_v = e_v + jnp.where(sc >= base_tile[e] * G, 1, 0)
            bs_v = zi
            bt_v = zi
            for e in range(E):
                sel = e_v == e
                bs_v = jnp.where(sel, base_tile[e] * G, bs_v)
                bt_v = jnp.where(sel, base_tile[e], bt_v)
            loc = sc - bs_v
            t_v = loc >> 2
            g_v = loc & 3
            seb[pl.ds(ch * 16, 16)] = e_v
            sgb[pl.ds(ch * 16, 16)] = g_v
            stxb[pl.ds(ch * 16, 16)] = bt_v + t_v
            sftb[pl.ds(ch * 16, 16)] = jnp.where(t_v == 0, 1, 0)
        btb[...] = jnp.where(lane == 0, nsteps, 0)
        pltpu.sync_copy(seb, se_hbm)
        pltpu.sync_copy(sgb, sg_hbm)
        pltpu.sync_copy(stxb, stx_hbm)
        pltpu.sync_copy(sftb, sft_hbm)
        pltpu.sync_copy(btb, bt_hbm)


def _assign(es, ws, cnt, x):
    mesh = plsc.VectorSubcoreMesh(core_axis_name="c", subcore_axis_name="s",
                                  num_cores=2, num_subcores=16)
    f = functools.partial(
        pl.kernel,
        out_type=[
            jax.ShapeDtypeStruct((P,), jnp.float32),
            jax.ShapeDtypeStruct((2 * N,), jnp.int32),
            jax.ShapeDtypeStruct((16,), jnp.int32),
            jax.ShapeDtypeStruct((MAXS,), jnp.int32),
            jax.ShapeDtypeStruct((MAXS,), jnp.int32),
            jax.ShapeDtypeStruct((MAXS,), jnp.int32),
            jax.ShapeDtypeStruct((MAXS,), jnp.int32),
            jax.ShapeDtypeStruct((P, D), jnp.float32),
        ],
        mesh=mesh,
        scratch_types=[
            pltpu.VMEM((_RC,), jnp.int32),
            pltpu.VMEM((_RNC, _RCH), jnp.float32),
            pltpu.VMEM((_RNC, _RCH), jnp.int32),
            pltpu.VMEM((_RNC, _RCH), jnp.int32),
            pltpu.VMEM((_RNC, _RCH), jnp.int32),
            pltpu.VMEM((NW, 16), jnp.int32),
            pltpu.VMEM((16,), jnp.int32),
            pltpu.VMEM((MAXS,), jnp.int32),
            pltpu.VMEM((MAXS,), jnp.int32),
            pltpu.VMEM((MAXS,), jnp.int32),
            pltpu.VMEM((MAXS,), jnp.int32),
            pltpu.VMEM((_RCH, D), jnp.float32),
            pltpu.VMEM((_RCH, D), jnp.float32),
            pltpu.VMEM((_RCH, D), jnp.float32),
            pltpu.SemaphoreType.DMA,
            pltpu.SemaphoreType.DMA,
            pltpu.SemaphoreType.DMA,
            pltpu.SemaphoreType.DMA,
            pltpu.SemaphoreType.DMA,
            pltpu.SemaphoreType.DMA,
            pltpu.SemaphoreType.DMA,
        ],
        compiler_params=pltpu.CompilerParams(needs_layout_passes=False),
    )
    return f(_assign_body)(es, ws, cnt, x)


MAXS = NTT * G


def _gffn_body(nst_ref, se_ref, sg_ref, stx_ref, sft_ref, xg_ref, w1_ref,
               b1_ref, w2_ref, b2_ref, w3_ref, b3_ref, pw_ref, out_ref,
               h1_ref, w1b_ref, w2b_ref, w3b_ref):
    s = pl.program_id(0)
    live = s < nst_ref[0]
    g = sg_ref[s]
    ft = sft_ref[s]

    @pl.when(live & (ft == 1))
    def _():
        w2b_ref[g] = w2_ref[0].astype(jnp.bfloat16)
        w3b_ref[g] = w3_ref[0].astype(jnp.bfloat16)
        @pl.when(g == 0)
        def _():
            w1b_ref[...] = w1_ref[0].astype(jnp.bfloat16)

    @pl.when(live)
    def _():
        @pl.when(g == 0)
        def _():
            h1 = jax.lax.dot_general(
                xg_ref[...].astype(jnp.bfloat16), w1b_ref[...],
                (((1,), (1,)), ((), ())),
                preferred_element_type=jnp.float32) + b1_ref[0]
            h1_ref[...] = jnp.maximum(h1, 0.0).astype(jnp.bfloat16)
        h2c = jax.lax.dot_general(
            h1_ref[...], w2b_ref[g], (((1,), (1,)), ((), ())),
            preferred_element_type=jnp.float32) + b2_ref[0]
        h2c = jnp.maximum(h2c, 0.0).astype(jnp.bfloat16)
        part = jax.lax.dot_general(
            h2c, w3b_ref[g], (((1,), (1,)), ((), ())),
            preferred_element_type=jnp.float32)
        @pl.when(g == 0)
        def _():
            out_ref[...] = part + b3_ref[0]
        @pl.when((g > 0) & (g < G - 1))
        def _():
            out_ref[...] += part
        @pl.when(g == G - 1)
        def _():
            out_ref[...] = (out_ref[...] + part) * pw_ref[0]


def _gffn(nst, se, sg, stx, sft, xg, W1, b1, W2, b2, W3, b3, pw):
    b1r = b1.reshape(E, 1, H)
    b2r = b2.reshape(E, 1, H)
    b3r = b3.reshape(E, 1, O)
    pw3 = pw.reshape(NTT, TN, 1)
    grid_spec = pltpu.PrefetchScalarGridSpec(
        num_scalar_prefetch=5,
        grid=(MAXS,),
        in_specs=[
            pl.BlockSpec((TN, D),
                         lambda s, nst, se, sg, stx, sft: (stx[s], 0)),
            pl.BlockSpec((1, H, D),
                         lambda s, nst, se, sg, stx, sft: (se[s], 0, 0)),
            pl.BlockSpec((1, 1, H),
                         lambda s, nst, se, sg, stx, sft: (se[s], 0, 0)),
            pl.BlockSpec((1, HC, H),
                         lambda s, nst, se, sg, stx, sft:
                         (se[s], jnp.where(sft[s] == 1, sg[s], G - 1), 0)),
            pl.BlockSpec((1, 1, HC),
                         lambda s, nst, se, sg, stx, sft: (se[s], 0, sg[s])),
            pl.BlockSpec((1, O, HC),
                         lambda s, nst, se, sg, stx, sft:
                         (se[s], 0, jnp.where(sft[s] == 1, sg[s], G - 1))),
            pl.BlockSpec((1, 1, O),
                         lambda s, nst, se, sg, stx, sft: (se[s], 0, 0)),
            pl.BlockSpec((1, TN, 1),
                         lambda s, nst, se, sg, stx, sft: (stx[s], 0, 0)),
        ],
        out_specs=pl.BlockSpec((TN, O),
                               lambda s, nst, se, sg, stx, sft: (stx[s], 0)),
        scratch_shapes=[
            pltpu.VMEM((TN, H), jnp.bfloat16),
            pltpu.VMEM((H, D), jnp.bfloat16),
            pltpu.VMEM((G, HC, H), jnp.bfloat16),
            pltpu.VMEM((G, O, HC), jnp.bfloat16),
        ],
    )
    return pl.pallas_call(
        _gffn_body,
        grid_spec=grid_spec,
        out_shape=jax.ShapeDtypeStruct((P, O), jnp.float32),
    )(nst, se, sg, stx, sft, xg, W1, b1r, W2, b2r, W3, b3r, pw3)


def _combine_body(ys_hbm, pos2_hbm, out_hbm, p0_v, p1_v, r0a, r1a, r0b, r1b,
                  sga0, sga1, sgb0, sgb1, swa, swb):
    wid = lax.axis_index("s") * 2 + lax.axis_index("c")
    base_t = wid * _CT
    nch = _CT // _CCH
    pltpu.sync_copy(pos2_hbm.at[pl.ds(base_t, _CT)], p0_v)
    pltpu.sync_copy(pos2_hbm.at[pl.ds(N + base_t, _CT)], p1_v)
    r0s = (r0a, r0b)
    r1s = (r1a, r1b)
    gs0 = (sga0, sgb0)
    gs1 = (sga1, sgb1)
    ws = (swa, swb)

    def start_pair(c):
        sl = pl.ds(c * _CCH, _CCH)
        return (pltpu.async_copy(ys_hbm.at[p0_v.at[sl]], r0s[c % 2],
                                 gs0[c % 2]),
                pltpu.async_copy(ys_hbm.at[p1_v.at[sl]], r1s[c % 2],
                                 gs1[c % 2]))

    g = {0: start_pair(0), 1: start_pair(1)}
    w = {}
    for c in range(nch):
        g[c][0].wait()
        g[c][1].wait()
        r0_v = r0s[c % 2]
        r1_v = r1s[c % 2]

        def body(r, _):
            for cc in range(O // 16):
                sl = pl.ds(cc * 16, 16)
                r0_v[r, sl] += r1_v[r, sl]
            return 0

        lax.fori_loop(0, _CCH, body, 0)
        w[c] = pltpu.async_copy(r0_v, out_hbm.at[pl.ds(base_t + c * _CCH,
                                                       _CCH)], ws[c % 2])
        if c + 2 < nch:
            w[c].wait()
            g[c + 2] = start_pair(c + 2)
    for c in range(max(0, nch - 2), nch):
        w[c].wait()


def _combine(ys, pos2):
    mesh = plsc.VectorSubcoreMesh(core_axis_name="c", subcore_axis_name="s",
                                  num_cores=2, num_subcores=16)
    f = functools.partial(
        pl.kernel,
        out_type=jax.ShapeDtypeStruct((N, O), jnp.float32),
        mesh=mesh,
        scratch_types=[
            pltpu.VMEM((_CT,), jnp.int32),
            pltpu.VMEM((_CT,), jnp.int32),
            pltpu.VMEM((_CCH, O), jnp.float32),
            pltpu.VMEM((_CCH, O), jnp.float32),
            pltpu.VMEM((_CCH, O), jnp.float32),
            pltpu.VMEM((_CCH, O), jnp.float32),
            pltpu.SemaphoreType.DMA,
            pltpu.SemaphoreType.DMA,
            pltpu.SemaphoreType.DMA,
            pltpu.SemaphoreType.DMA,
            pltpu.SemaphoreType.DMA,
            pltpu.SemaphoreType.DMA,
        ],
    )
    return f(_combine_body)(ys, pos2)


def kernel(x, Wg, bg, W1, b1, W2, b2, W3, b3):
    probs_p, idx_p, wn_p, loss2 = _gating(x, Wg, bg)
    gate_probs = probs_p[:, :E]
    loss = loss2.reshape(())
    es = idx_p[:, :TOPK].reshape(S)
    ws = wn_p[:, :TOPK].reshape(S)

    cnt = _count(es)
    pw, pos2, nst, se, sg, stx, sft, xg = _assign(es, ws, cnt, x)
    ys = _gffn(nst, se, sg, stx, sft, xg, W1, b1, W2, b2, W3, b3, pw)
    final = _combine(ys, pos2)
    return (final, loss, gate_probs)
